# Initial kernel scaffold; baseline (speedup 1.0000x reference)
import jax
import jax.numpy as jnp
from jax.experimental import pallas as pl


def kernel(x, edge_index, batch, dropout, edge_attr, Wl1, Wr1, We1, att1, b1, Wl2, Wr2, We2, att2, b2, Wlin, blin):
    raise NotImplementedError("write your pallas kernel here")


# speedup vs baseline: 7.9887x; 7.9887x over previous
import functools

import jax
import jax.numpy as jnp
from jax import lax
from jax.experimental import pallas as pl
from jax.experimental.pallas import tpu as pltpu
from jax.experimental.pallas import tpu_sc as plsc

_N = 10000
_NP = 10240
_E = 320000
_D = 128
_DE = 16
_NG = 64
_CH = 64
_RPT = _NP // 16


def _worker_ids():
    sid = lax.axis_index("s")
    cid = lax.axis_index("c")
    return sid, cid


def _chunk_split(chunks_per_sc, sid):
    q, r = divmod(chunks_per_sc, 16)
    nw = q + jnp.where(sid < r, 1, 0)
    start = q * sid + jnp.minimum(sid, r)
    return nw, start


def _zero16():
    return jnp.zeros((16,), jnp.float32)


@functools.partial(
    pl.kernel,
    out_type=jax.ShapeDtypeStruct((2, _NP, _D), jnp.float32),
    mesh=plsc.VectorSubcoreMesh(core_axis_name="c", subcore_axis_name="s"),
    compiler_params=pltpu.CompilerParams(needs_layout_passes=False),
    scratch_types=[
        pltpu.VMEM((_CH,), jnp.int32),
        pltpu.VMEM((_CH, _DE), jnp.float32),
        pltpu.VMEM((_CH, _D), jnp.float32),
        pltpu.VMEM_SHARED((_NP, _D), jnp.float32),
        pltpu.SemaphoreType.DMA,
    ],
)
def _deg_attr_pass(dst_hbm, ea_hbm, attr_out, dst_v, ea_v, pad_v, attr_sh,
                   sem):
    sid, cid = _worker_ids()
    lane = lax.iota(jnp.int32, 16)

    def zrows(i, c):
        for j in range(_D // 16):
            pad_v[i, pl.ds(16 * j, 16)] = _zero16()
        return c
    lax.fori_loop(0, _CH, zrows, 0)
    for p in range(_RPT // _CH):
        pltpu.sync_copy(pad_v, attr_sh.at[pl.ds(sid * _RPT + p * _CH, _CH)])

    def ones_col(i, c):
        pad_v[i, pl.ds(16, 16)] = jnp.where(lane == 0, 1.0, 0.0)
        return c
    lax.fori_loop(0, _CH, ones_col, 0)
    plsc.subcore_barrier()

    chunks_per_sc = (_E // 2) // _CH
    nw, start = _chunk_split(chunks_per_sc, sid)

    def body(k, c):
        base = cid * (_E // 2) + (start + k) * _CH
        pltpu.sync_copy(dst_hbm.at[pl.ds(base, _CH)], dst_v)
        pltpu.sync_copy(ea_hbm.at[pl.ds(base, _CH)], ea_v)

        def fill(i, c2):
            pad_v[i, pl.ds(0, 16)] = ea_v[i, :]
            return c2
        lax.fori_loop(0, _CH, fill, 0)
        pltpu.sync_copy(pad_v, attr_sh.at[dst_v], add=True)
        return c
    lax.fori_loop(0, nw, body, 0)

    plsc.subcore_barrier()
    pltpu.sync_copy(attr_sh.at[pl.ds(sid * _RPT, _RPT)],
                    attr_out.at[cid, pl.ds(sid * _RPT, _RPT)])


@functools.partial(
    pl.kernel,
    out_type=[
        jax.ShapeDtypeStruct((2, 16, _NP), jnp.float32),
        jax.ShapeDtypeStruct((2, _NP, _D), jnp.float32),
    ],
    mesh=plsc.VectorSubcoreMesh(core_axis_name="c", subcore_axis_name="s"),
    compiler_params=pltpu.CompilerParams(needs_layout_passes=False),
    scratch_types=[
        pltpu.VMEM((_NP,), jnp.float32),
        pltpu.VMEM((_NP,), jnp.float32),
        pltpu.VMEM((_D,), jnp.float32),
        pltpu.VMEM((_CH,), jnp.int32),
        pltpu.VMEM((_CH,), jnp.int32),
        pltpu.VMEM((_CH, _D), jnp.float32),
        pltpu.VMEM((_CH, _D), jnp.float32),
        pltpu.VMEM((_CH, _D), jnp.float32),
        pltpu.VMEM_SHARED((_NP, _D), jnp.float32),
        pltpu.SemaphoreType.DMA,
        pltpu.SemaphoreType.DMA,
        pltpu.SemaphoreType.DMA,
    ],
)
def _edge_pass(src_hbm, dst_hbm, ee_hbm, xl_hbm, xr_hbm, shift_hbm, att_hbm,
               den_out, acc_out,
               shf_v, den_v, att_v, src_v, dst_v, ee_v, xl_v, xr_v,
               acc_sh, sem1, sem2, sem3):
    sid, cid = _worker_ids()
    lane = lax.iota(jnp.int32, 16)

    def zrows(i, c):
        for j in range(_D // 16):
            ee_v[i, pl.ds(16 * j, 16)] = _zero16()
        return c
    lax.fori_loop(0, _CH, zrows, 0)
    for p in range(_RPT // _CH):
        pltpu.sync_copy(ee_v, acc_sh.at[pl.ds(sid * _RPT + p * _CH, _CH)])

    def zden(i, c):
        den_v[pl.ds(16 * i, 16)] = _zero16()
        return c
    lax.fori_loop(0, _NP // 16, zden, 0)

    pltpu.sync_copy(shift_hbm, shf_v)
    pltpu.sync_copy(att_hbm, att_v)
    plsc.subcore_barrier()

    chunks_per_sc = (_E // 2) // _CH
    nw, start = _chunk_split(chunks_per_sc, sid)

    def body(k, c):
        base = cid * (_E // 2) + (start + k) * _CH
        pltpu.sync_copy(src_hbm.at[pl.ds(base, _CH)], src_v)
        pltpu.sync_copy(dst_hbm.at[pl.ds(base, _CH)], dst_v)
        cp1 = pltpu.async_copy(xl_hbm.at[src_v], xl_v, sem1)
        cp2 = pltpu.async_copy(xr_hbm.at[dst_v], xr_v, sem2)
        cp3 = pltpu.async_copy(ee_hbm.at[pl.ds(base, _CH)], ee_v, sem3)
        cp1.wait()
        cp2.wait()
        cp3.wait()

        def group_body(g, c2):
            gsl = pl.ds(16 * g, 16)
            d16 = dst_v[gsl]
            sh16 = plsc.load_gather(shf_v, [d16])
            acc16 = _zero16()
            for e in range(16):
                ei = 16 * g + e
                a = _zero16()
                for j in range(_D // 16):
                    sl = pl.ds(16 * j, 16)
                    z = xl_v[ei, sl] + xr_v[ei, sl] + ee_v[ei, sl]
                    z = jnp.where(z >= 0.0, z, 0.2 * z)
                    a = a + z * att_v[sl]
                acc16 = jnp.where(lane == e, jnp.sum(a), acc16)
            ex16 = jnp.exp(acc16 - sh16)
            for e in range(16):
                plsc.addupdate_scatter(den_v, [d16], ex16, mask=lane == e)
            for e in range(16):
                ei = 16 * g + e
                w = jnp.sum(jnp.where(lane == e, ex16, 0.0))
                for j in range(_D // 16):
                    sl = pl.ds(16 * j, 16)
                    xl_v[ei, sl] = xl_v[ei, sl] * w
            return c2
        lax.fori_loop(0, _CH // 16, group_body, 0)

        pltpu.sync_copy(xl_v, acc_sh.at[dst_v], add=True)
        return c
    lax.fori_loop(0, nw, body, 0)

    pltpu.sync_copy(den_v, den_out.at[cid, sid])
    plsc.subcore_barrier()
    pltpu.sync_copy(acc_sh.at[pl.ds(sid * _RPT, _RPT)],
                    acc_out.at[cid, pl.ds(sid * _RPT, _RPT)])


_RB = 2000


def _mm2_body(x_ref, wl_ref, wr_ref, xl_ref, xr_ref):
    xb = x_ref[...]
    xl_ref[...] = jnp.dot(xb, wl_ref[...], preferred_element_type=jnp.float32)
    xr_ref[...] = jnp.dot(xb, wr_ref[...], preferred_element_type=jnp.float32)


def _mm2(x, wl, wr):
    return pl.pallas_call(
        _mm2_body,
        grid=(_N // _RB,),
        in_specs=[
            pl.BlockSpec((_RB, _D), lambda i: (i, 0)),
            pl.BlockSpec((_D, _D), lambda i: (0, 0)),
            pl.BlockSpec((_D, _D), lambda i: (0, 0)),
        ],
        out_specs=[pl.BlockSpec((_RB, _D), lambda i: (i, 0))] * 2,
        out_shape=[jax.ShapeDtypeStruct((_N, _D), jnp.float32)] * 2,
    )(x, wl, wr)


_EB = 8000


def _ee_body(ea_ref, we_ref, ee_ref):
    ee_ref[...] = jnp.dot(ea_ref[...], we_ref[...],
                          preferred_element_type=jnp.float32)


def _ee_mm(ea, we):
    return pl.pallas_call(
        _ee_body,
        grid=(_E // _EB,),
        in_specs=[
            pl.BlockSpec((_EB, _DE), lambda i: (i, 0)),
            pl.BlockSpec((_DE, _D), lambda i: (0, 0)),
        ],
        out_specs=pl.BlockSpec((_EB, _D), lambda i: (i, 0)),
        out_shape=jax.ShapeDtypeStruct((_E, _D), jnp.float32),
    )(ea, we)


def _shift_body(xl_ref, xr_ref, attr_ref, we_ref, att_ref, s_ref):
    deg = attr_ref[0, :, 16:17] + attr_ref[1, :, 16:17]
    ma = (attr_ref[0, :, :_DE] + attr_ref[1, :, :_DE]) / jnp.maximum(deg, 1.0)
    z = xl_ref[...] + xr_ref[...] + jnp.dot(
        ma, we_ref[...], preferred_element_type=jnp.float32)
    z = jnp.where(z >= 0.0, z, 0.2 * z)
    s_ref[...] = jnp.dot(z, att_ref[...], preferred_element_type=jnp.float32)


def _shift(xl, xr, attrdeg, we, att):
    s = pl.pallas_call(
        _shift_body,
        grid=(_N // _RB,),
        in_specs=[
            pl.BlockSpec((_RB, _D), lambda i: (i, 0)),
            pl.BlockSpec((_RB, _D), lambda i: (i, 0)),
            pl.BlockSpec((2, _RB, _D), lambda i: (0, i, 0)),
            pl.BlockSpec((_DE, _D), lambda i: (0, 0)),
            pl.BlockSpec((_D, 1), lambda i: (0, 0)),
        ],
        out_specs=pl.BlockSpec((_RB, 1), lambda i: (i, 0)),
        out_shape=jax.ShapeDtypeStruct((_N, 1), jnp.float32),
    )(xl, xr, attrdeg, we, att.reshape(_D, 1))
    return jnp.pad(s.reshape(_N), (0, _NP - _N))


def _combine_body(acc_ref, den_ref, xl_ref, b_ref, h_ref):
    a = acc_ref[0] + acc_ref[1] + xl_ref[...]
    dn = jnp.sum(den_ref[...], axis=1, keepdims=True) + 1.0
    h = a / dn + b_ref[...]
    h_ref[...] = jnp.maximum(h, 0.0)


def _combine(acc, den, xl, b):
    return pl.pallas_call(
        _combine_body,
        grid=(_N // _RB,),
        in_specs=[
            pl.BlockSpec((2, _RB, _D), lambda i: (0, i, 0)),
            pl.BlockSpec((_RB, 32), lambda i: (i, 0)),
            pl.BlockSpec((_RB, _D), lambda i: (i, 0)),
            pl.BlockSpec((1, _D), lambda i: (0, 0)),
        ],
        out_specs=pl.BlockSpec((_RB, _D), lambda i: (i, 0)),
        out_shape=jax.ShapeDtypeStruct((_N, _D), jnp.float32),
    )(acc, den, xl, b.reshape(1, _D))


def _pool_body(h_ref, batch_ref, wlin_ref, blin_ref, o_ref):
    b = batch_ref[...]
    g = lax.broadcasted_iota(jnp.int32, (_NG, 1), 0)
    oh = (b == g).astype(jnp.float32)
    cnt = jnp.sum(oh, axis=1, keepdims=True)
    ps = jnp.dot(oh, h_ref[...], preferred_element_type=jnp.float32)
    pooled = ps / jnp.maximum(cnt, 1.0)
    o_ref[...] = jnp.dot(pooled, wlin_ref[...],
                         preferred_element_type=jnp.float32) + blin_ref[...]


def _pool(h, batch, wlin, blin):
    nc = wlin.shape[1]
    return pl.pallas_call(
        _pool_body,
        out_shape=jax.ShapeDtypeStruct((_NG, nc), jnp.float32),
    )(h, batch.reshape(1, _N), wlin, blin.reshape(1, nc))


def _layer(x, src, dst, ea, attrdeg, wl, wr, we, att, b):
    xl, xr = _mm2(x, wl, wr)
    ee = _ee_mm(ea, we)
    sh = _shift(xl, xr, attrdeg, we, att)
    den, acc = _edge_pass(src, dst, ee, xl, xr, sh, att)
    den32 = den.reshape(32, _NP)[:, :_N].T
    return _combine(acc, den32, xl, b)


def kernel(x, edge_index, batch, dropout, edge_attr, Wl1, Wr1, We1, att1, b1,
           Wl2, Wr2, We2, att2, b2, Wlin, blin):
    src = edge_index[0]
    dst = edge_index[1]
    attrdeg = _deg_attr_pass(dst, edge_attr)
    h = _layer(x, src, dst, edge_attr, attrdeg, Wl1, Wr1, We1, att1, b1)
    h = _layer(h, src, dst, edge_attr, attrdeg, Wl2, Wr2, We2, att2, b2)
    return _pool(h, batch, Wlin, blin)


# speedup vs baseline: 9.1106x; 1.1404x over previous
import functools

import jax
import jax.numpy as jnp
from jax import lax
from jax.experimental import pallas as pl
from jax.experimental.pallas import tpu as pltpu
from jax.experimental.pallas import tpu_sc as plsc

_N = 10000
_NP = 10240
_E = 320000
_D = 128
_DE = 16
_NG = 64
_CH = 32
_RPT = _NP // 16


def _worker_ids():
    sid = lax.axis_index("s")
    cid = lax.axis_index("c")
    return sid, cid


def _chunk_split(chunks_per_sc, sid):
    q, r = divmod(chunks_per_sc, 16)
    nw = q + jnp.where(sid < r, 1, 0)
    start = q * sid + jnp.minimum(sid, r)
    return nw, start


def _zero16():
    return jnp.zeros((16,), jnp.float32)


@functools.partial(
    pl.kernel,
    out_type=jax.ShapeDtypeStruct((2, _NP, _D), jnp.float32),
    mesh=plsc.VectorSubcoreMesh(core_axis_name="c", subcore_axis_name="s"),
    compiler_params=pltpu.CompilerParams(needs_layout_passes=False),
    scratch_types=[
        pltpu.VMEM((_CH,), jnp.int32),
        pltpu.VMEM((_CH,), jnp.int32),
        pltpu.VMEM((_CH, _DE), jnp.float32),
        pltpu.VMEM((_CH, _DE), jnp.float32),
        pltpu.VMEM((_CH, _D), jnp.float32),
        pltpu.VMEM((_CH, _D), jnp.float32),
        pltpu.VMEM_SHARED((_NP, _D), jnp.float32),
        pltpu.SemaphoreType.DMA,
        pltpu.SemaphoreType.DMA,
    ],
)
def _deg_attr_pass(dst_hbm, ea_hbm, attr_out, dst0, dst1, ea0, ea1,
                   pad0, pad1, attr_sh, sem0, sem1):
    sid, cid = _worker_ids()
    lane = lax.iota(jnp.int32, 16)
    dstb, eab, padb, semb = (dst0, dst1), (ea0, ea1), (pad0, pad1), (sem0, sem1)

    def zrows(i, c):
        for j in range(_D // 16):
            pad0[i, pl.ds(16 * j, 16)] = _zero16()
            pad1[i, pl.ds(16 * j, 16)] = _zero16()
        return c
    lax.fori_loop(0, _CH, zrows, 0)
    for p in range(_RPT // _CH):
        pltpu.sync_copy(pad0, attr_sh.at[pl.ds(sid * _RPT + p * _CH, _CH)])

    def ones_col(i, c):
        pad0[i, pl.ds(16, 16)] = jnp.where(lane == 0, 1.0, 0.0)
        pad1[i, pl.ds(16, 16)] = jnp.where(lane == 0, 1.0, 0.0)
        return c
    lax.fori_loop(0, _CH, ones_col, 0)
    plsc.subcore_barrier()

    chunks_per_sc = (_E // 2) // _CH
    nw, start = _chunk_split(chunks_per_sc, sid)
    ebase = cid * (_E // 2)

    def issue(k, b):
        base = ebase + (start + k) * _CH
        pltpu.sync_copy(dst_hbm.at[pl.ds(base, _CH)], dstb[b])
        pltpu.async_copy(ea_hbm.at[pl.ds(base, _CH)], eab[b], semb[b])

    issue(0, 0)

    def outer(kk, c):
        for b in range(2):
            k = 2 * kk + b

            @pl.when(k < nw)
            def _():
                @pl.when(k + 1 < nw)
                def _():
                    issue(k + 1, 1 - b)
                base = ebase + (start + k) * _CH
                pltpu.make_async_copy(
                    ea_hbm.at[pl.ds(base, _CH)], eab[b], semb[b]).wait()

                def fill(i, c2):
                    padb[b][i, pl.ds(0, 16)] = eab[b][i, :]
                    return c2
                lax.fori_loop(0, _CH, fill, 0)
                pltpu.sync_copy(padb[b], attr_sh.at[dstb[b]], add=True)
        return c
    lax.fori_loop(0, (nw + 1) // 2, outer, 0)

    plsc.subcore_barrier()
    pltpu.sync_copy(attr_sh.at[pl.ds(sid * _RPT, _RPT)],
                    attr_out.at[cid, pl.ds(sid * _RPT, _RPT)])


@functools.partial(
    pl.kernel,
    out_type=[
        jax.ShapeDtypeStruct((2, 16, _NP), jnp.float32),
        jax.ShapeDtypeStruct((2, _NP, _D), jnp.float32),
    ],
    mesh=plsc.VectorSubcoreMesh(core_axis_name="c", subcore_axis_name="s"),
    compiler_params=pltpu.CompilerParams(needs_layout_passes=False),
    scratch_types=[
        pltpu.VMEM((_NP,), jnp.float32),
        pltpu.VMEM((_NP,), jnp.float32),
        pltpu.VMEM((_D,), jnp.float32),
        pltpu.VMEM((_CH,), jnp.int32),
        pltpu.VMEM((_CH,), jnp.int32),
        pltpu.VMEM((_CH,), jnp.int32),
        pltpu.VMEM((_CH,), jnp.int32),
        pltpu.VMEM((_CH, _D), jnp.float32),
        pltpu.VMEM((_CH, _D), jnp.float32),
        pltpu.VMEM((_CH, _D), jnp.float32),
        pltpu.VMEM((_CH, _D), jnp.float32),
        pltpu.VMEM((_CH, _D), jnp.float32),
        pltpu.VMEM((_CH, _D), jnp.float32),
        pltpu.VMEM_SHARED((_NP, _D), jnp.float32),
        pltpu.SemaphoreType.DMA,
        pltpu.SemaphoreType.DMA,
    ],
)
def _edge_pass(src_hbm, dst_hbm, ee_hbm, xl_hbm, xr_hbm, shift_hbm, att_hbm,
               den_out, acc_out,
               shf_v, den_v, att_v, src0, src1, dst0, dst1, ee0, ee1,
               xl0, xl1, xr0, xr1, acc_sh, sem0, sem1):
    sid, cid = _worker_ids()
    lane = lax.iota(jnp.int32, 16)
    srcb, dstb = (src0, src1), (dst0, dst1)
    eeb, xlb, xrb, semb = (ee0, ee1), (xl0, xl1), (xr0, xr1), (sem0, sem1)

    def zrows(i, c):
        for j in range(_D // 16):
            ee0[i, pl.ds(16 * j, 16)] = _zero16()
        return c
    lax.fori_loop(0, _CH, zrows, 0)
    for p in range(_RPT // _CH):
        pltpu.sync_copy(ee0, acc_sh.at[pl.ds(sid * _RPT + p * _CH, _CH)])

    def zden(i, c):
        den_v[pl.ds(16 * i, 16)] = _zero16()
        return c
    lax.fori_loop(0, _NP // 16, zden, 0)

    pltpu.sync_copy(shift_hbm, shf_v)
    pltpu.sync_copy(att_hbm, att_v)
    plsc.subcore_barrier()

    chunks_per_sc = (_E // 2) // _CH
    nw, start = _chunk_split(chunks_per_sc, sid)
    ebase = cid * (_E // 2)

    def issue(k, b):
        base = ebase + (start + k) * _CH
        pltpu.sync_copy(src_hbm.at[pl.ds(base, _CH)], srcb[b])
        pltpu.sync_copy(dst_hbm.at[pl.ds(base, _CH)], dstb[b])
        pltpu.async_copy(xl_hbm.at[srcb[b]], xlb[b], semb[b])
        pltpu.async_copy(xr_hbm.at[dstb[b]], xrb[b], semb[b])
        pltpu.async_copy(ee_hbm.at[pl.ds(base, _CH)], eeb[b], semb[b])

    issue(0, 0)

    def outer(kk, c):
        for b in range(2):
            k = 2 * kk + b

            @pl.when(k < nw)
            def _():
                @pl.when(k + 1 < nw)
                def _():
                    issue(k + 1, 1 - b)
                base = ebase + (start + k) * _CH
                pltpu.make_async_copy(
                    xl_hbm.at[srcb[b]], xlb[b], semb[b]).wait()
                pltpu.make_async_copy(
                    xr_hbm.at[dstb[b]], xrb[b], semb[b]).wait()
                pltpu.make_async_copy(
                    ee_hbm.at[pl.ds(base, _CH)], eeb[b], semb[b]).wait()

                def group_body(g, c2):
                    gsl = pl.ds(16 * g, 16)
                    d16 = dstb[b][gsl]
                    sh16 = plsc.load_gather(shf_v, [d16])
                    acc16 = _zero16()
                    for e in range(16):
                        ei = 16 * g + e
                        a = _zero16()
                        for j in range(_D // 16):
                            sl = pl.ds(16 * j, 16)
                            z = xlb[b][ei, sl] + xrb[b][ei, sl] + eeb[b][ei, sl]
                            z = jnp.where(z >= 0.0, z, 0.2 * z)
                            a = a + z * att_v[sl]
                        acc16 = jnp.where(lane == e, jnp.sum(a), acc16)
                    ex16 = jnp.exp(acc16 - sh16)
                    for e in range(16):
                        plsc.addupdate_scatter(den_v, [d16], ex16,
                                               mask=lane == e)
                    for e in range(16):
                        ei = 16 * g + e
                        w = jnp.sum(jnp.where(lane == e, ex16, 0.0))
                        for j in range(_D // 16):
                            sl = pl.ds(16 * j, 16)
                            xlb[b][ei, sl] = xlb[b][ei, sl] * w
                    return c2
                lax.fori_loop(0, _CH // 16, group_body, 0)

                pltpu.sync_copy(xlb[b], acc_sh.at[dstb[b]], add=True)
        return c
    lax.fori_loop(0, (nw + 1) // 2, outer, 0)

    pltpu.sync_copy(den_v, den_out.at[cid, sid])
    plsc.subcore_barrier()
    pltpu.sync_copy(acc_sh.at[pl.ds(sid * _RPT, _RPT)],
                    acc_out.at[cid, pl.ds(sid * _RPT, _RPT)])


_RB = 2000


def _mm2_body(x_ref, wl_ref, wr_ref, xl_ref, xr_ref):
    xb = x_ref[...]
    xl_ref[...] = jnp.dot(xb, wl_ref[...], preferred_element_type=jnp.float32)
    xr_ref[...] = jnp.dot(xb, wr_ref[...], preferred_element_type=jnp.float32)


def _mm2(x, wl, wr):
    return pl.pallas_call(
        _mm2_body,
        grid=(_N // _RB,),
        in_specs=[
            pl.BlockSpec((_RB, _D), lambda i: (i, 0)),
            pl.BlockSpec((_D, _D), lambda i: (0, 0)),
            pl.BlockSpec((_D, _D), lambda i: (0, 0)),
        ],
        out_specs=[pl.BlockSpec((_RB, _D), lambda i: (i, 0))] * 2,
        out_shape=[jax.ShapeDtypeStruct((_N, _D), jnp.float32)] * 2,
    )(x, wl, wr)


_EB = 8000


def _ee_body(ea_ref, we_ref, ee_ref):
    ee_ref[...] = jnp.dot(ea_ref[...], we_ref[...],
                          preferred_element_type=jnp.float32)


def _ee_mm(ea, we):
    return pl.pallas_call(
        _ee_body,
        grid=(_E // _EB,),
        in_specs=[
            pl.BlockSpec((_EB, _DE), lambda i: (i, 0)),
            pl.BlockSpec((_DE, _D), lambda i: (0, 0)),
        ],
        out_specs=pl.BlockSpec((_EB, _D), lambda i: (i, 0)),
        out_shape=jax.ShapeDtypeStruct((_E, _D), jnp.float32),
    )(ea, we)


def _shift_body(xl_ref, xr_ref, attr_ref, we_ref, att_ref, s_ref):
    deg = attr_ref[0, :, 16:17] + attr_ref[1, :, 16:17]
    ma = (attr_ref[0, :, :_DE] + attr_ref[1, :, :_DE]) / jnp.maximum(deg, 1.0)
    z = xl_ref[...] + xr_ref[...] + jnp.dot(
        ma, we_ref[...], preferred_element_type=jnp.float32)
    z = jnp.where(z >= 0.0, z, 0.2 * z)
    s_ref[...] = jnp.dot(z, att_ref[...], preferred_element_type=jnp.float32)


def _shift(xl, xr, attrdeg, we, att):
    s = pl.pallas_call(
        _shift_body,
        grid=(_N // _RB,),
        in_specs=[
            pl.BlockSpec((_RB, _D), lambda i: (i, 0)),
            pl.BlockSpec((_RB, _D), lambda i: (i, 0)),
            pl.BlockSpec((2, _RB, _D), lambda i: (0, i, 0)),
            pl.BlockSpec((_DE, _D), lambda i: (0, 0)),
            pl.BlockSpec((_D, 1), lambda i: (0, 0)),
        ],
        out_specs=pl.BlockSpec((_RB, 1), lambda i: (i, 0)),
        out_shape=jax.ShapeDtypeStruct((_N, 1), jnp.float32),
    )(xl, xr, attrdeg, we, att.reshape(_D, 1))
    return jnp.pad(s.reshape(_N), (0, _NP - _N))


def _combine_body(acc_ref, den_ref, xl_ref, b_ref, h_ref):
    a = acc_ref[0] + acc_ref[1] + xl_ref[...]
    dn = jnp.sum(den_ref[...], axis=1, keepdims=True) + 1.0
    h = a / dn + b_ref[...]
    h_ref[...] = jnp.maximum(h, 0.0)


def _combine(acc, den, xl, b):
    return pl.pallas_call(
        _combine_body,
        grid=(_N // _RB,),
        in_specs=[
            pl.BlockSpec((2, _RB, _D), lambda i: (0, i, 0)),
            pl.BlockSpec((_RB, 32), lambda i: (i, 0)),
            pl.BlockSpec((_RB, _D), lambda i: (i, 0)),
            pl.BlockSpec((1, _D), lambda i: (0, 0)),
        ],
        out_specs=pl.BlockSpec((_RB, _D), lambda i: (i, 0)),
        out_shape=jax.ShapeDtypeStruct((_N, _D), jnp.float32),
    )(acc, den, xl, b.reshape(1, _D))


def _pool_body(h_ref, batch_ref, wlin_ref, blin_ref, o_ref):
    b = batch_ref[...]
    g = lax.broadcasted_iota(jnp.int32, (_NG, 1), 0)
    oh = (b == g).astype(jnp.float32)
    cnt = jnp.sum(oh, axis=1, keepdims=True)
    ps = jnp.dot(oh, h_ref[...], preferred_element_type=jnp.float32)
    pooled = ps / jnp.maximum(cnt, 1.0)
    o_ref[...] = jnp.dot(pooled, wlin_ref[...],
                         preferred_element_type=jnp.float32) + blin_ref[...]


def _pool(h, batch, wlin, blin):
    nc = wlin.shape[1]
    return pl.pallas_call(
        _pool_body,
        out_shape=jax.ShapeDtypeStruct((_NG, nc), jnp.float32),
    )(h, batch.reshape(1, _N), wlin, blin.reshape(1, nc))


def _layer(x, src, dst, ea, attrdeg, wl, wr, we, att, b):
    xl, xr = _mm2(x, wl, wr)
    ee = _ee_mm(ea, we)
    sh = _shift(xl, xr, attrdeg, we, att)
    den, acc = _edge_pass(src, dst, ee, xl, xr, sh, att)
    den32 = den.reshape(32, _NP)[:, :_N].T
    return _combine(acc, den32, xl, b)


def kernel(x, edge_index, batch, dropout, edge_attr, Wl1, Wr1, We1, att1, b1,
           Wl2, Wr2, We2, att2, b2, Wlin, blin):
    src = edge_index[0]
    dst = edge_index[1]
    attrdeg = _deg_attr_pass(dst, edge_attr)
    h = _layer(x, src, dst, edge_attr, attrdeg, Wl1, Wr1, We1, att1, b1)
    h = _layer(h, src, dst, edge_attr, attrdeg, Wl2, Wr2, We2, att2, b2)
    return _pool(h, batch, Wlin, blin)


# speedup vs baseline: 11.0244x; 1.2101x over previous
import functools

import jax
import jax.numpy as jnp
from jax import lax
from jax.experimental import pallas as pl
from jax.experimental.pallas import tpu as pltpu
from jax.experimental.pallas import tpu_sc as plsc

_N = 10000
_NP = 10240
_E = 320000
_D = 128
_DE = 16
_NG = 64
_CH = 32
_RPT = _NP // 16


def _worker_ids():
    sid = lax.axis_index("s")
    cid = lax.axis_index("c")
    return sid, cid


def _chunk_split(chunks_per_sc, sid):
    q, r = divmod(chunks_per_sc, 16)
    nw = q + jnp.where(sid < r, 1, 0)
    start = q * sid + jnp.minimum(sid, r)
    return nw, start


def _zero16():
    return jnp.zeros((16,), jnp.float32)


@functools.partial(
    pl.kernel,
    out_type=jax.ShapeDtypeStruct((2, _NP, _D), jnp.float32),
    mesh=plsc.VectorSubcoreMesh(core_axis_name="c", subcore_axis_name="s"),
    compiler_params=pltpu.CompilerParams(needs_layout_passes=False),
    scratch_types=[
        pltpu.VMEM((_CH,), jnp.int32),
        pltpu.VMEM((_CH,), jnp.int32),
        pltpu.VMEM((_CH, _DE), jnp.float32),
        pltpu.VMEM((_CH, _DE), jnp.float32),
        pltpu.VMEM((_CH, _D), jnp.float32),
        pltpu.VMEM((_CH, _D), jnp.float32),
        pltpu.VMEM_SHARED((_NP, _D), jnp.float32),
        pltpu.SemaphoreType.DMA,
        pltpu.SemaphoreType.DMA,
        pltpu.SemaphoreType.DMA,
        pltpu.SemaphoreType.DMA,
    ],
)
def _deg_attr_pass(dst_hbm, ea_hbm, attr_out, dst0, dst1, ea0, ea1,
                   pad0, pad1, attr_sh, sem0, sem1, semi0, semi1):
    sid, cid = _worker_ids()
    lane = lax.iota(jnp.int32, 16)
    dstb, eab, padb, semb = (dst0, dst1), (ea0, ea1), (pad0, pad1), (sem0, sem1)
    semib = (semi0, semi1)

    def zrows(i, c):
        for j in range(_D // 16):
            pad0[i, pl.ds(16 * j, 16)] = _zero16()
            pad1[i, pl.ds(16 * j, 16)] = _zero16()
        return c
    lax.fori_loop(0, _CH, zrows, 0)
    for p in range(_RPT // _CH):
        pltpu.sync_copy(pad0, attr_sh.at[pl.ds(sid * _RPT + p * _CH, _CH)])

    def ones_col(i, c):
        pad0[i, pl.ds(16, 16)] = jnp.where(lane == 0, 1.0, 0.0)
        pad1[i, pl.ds(16, 16)] = jnp.where(lane == 0, 1.0, 0.0)
        return c
    lax.fori_loop(0, _CH, ones_col, 0)
    plsc.subcore_barrier()

    chunks_per_sc = (_E // 2) // _CH
    nw, start = _chunk_split(chunks_per_sc, sid)
    ebase = cid * (_E // 2)

    def issue(k, b):
        base = ebase + (start + k) * _CH
        pltpu.async_copy(dst_hbm.at[pl.ds(base, _CH)], dstb[b], semib[b])
        pltpu.async_copy(ea_hbm.at[pl.ds(base, _CH)], eab[b], semb[b])

    def wait_in(k, b):
        base = ebase + (start + k) * _CH
        pltpu.make_async_copy(
            dst_hbm.at[pl.ds(base, _CH)], dstb[b], semib[b]).wait()
        pltpu.make_async_copy(
            ea_hbm.at[pl.ds(base, _CH)], eab[b], semb[b]).wait()

    issue(0, 0)

    @pl.when(nw > 1)
    def _():
        issue(1, 1)

    def outer(kk, c):
        for b in range(2):
            k = 2 * kk + b

            @pl.when(k < nw)
            def _():
                wait_in(k, b)

                def fill(i, c2):
                    padb[b][i, pl.ds(0, 16)] = eab[b][i, :]
                    return c2
                lax.fori_loop(0, _CH, fill, 0)
                pltpu.sync_copy(padb[b], attr_sh.at[dstb[b]], add=True)

                @pl.when(k + 2 < nw)
                def _():
                    issue(k + 2, b)
        return c
    lax.fori_loop(0, (nw + 1) // 2, outer, 0)

    plsc.subcore_barrier()
    pltpu.sync_copy(attr_sh.at[pl.ds(sid * _RPT, _RPT)],
                    attr_out.at[cid, pl.ds(sid * _RPT, _RPT)])


@functools.partial(
    pl.kernel,
    out_type=[
        jax.ShapeDtypeStruct((2, 16, _NP), jnp.float32),
        jax.ShapeDtypeStruct((2, _NP, _D), jnp.float32),
    ],
    mesh=plsc.VectorSubcoreMesh(core_axis_name="c", subcore_axis_name="s"),
    compiler_params=pltpu.CompilerParams(needs_layout_passes=False),
    scratch_types=[
        pltpu.VMEM((_NP,), jnp.float32),
        pltpu.VMEM((_NP,), jnp.float32),
        pltpu.VMEM((_D,), jnp.float32),
        pltpu.VMEM((_CH,), jnp.int32),
        pltpu.VMEM((_CH,), jnp.int32),
        pltpu.VMEM((_CH,), jnp.int32),
        pltpu.VMEM((_CH,), jnp.int32),
        pltpu.VMEM((_CH, _D), jnp.float32),
        pltpu.VMEM((_CH, _D), jnp.float32),
        pltpu.VMEM((_CH, _D), jnp.float32),
        pltpu.VMEM((_CH, _D), jnp.float32),
        pltpu.VMEM((_CH, _D), jnp.float32),
        pltpu.VMEM((_CH, _D), jnp.float32),
        pltpu.VMEM_SHARED((_NP, _D), jnp.float32),
        pltpu.SemaphoreType.DMA,
        pltpu.SemaphoreType.DMA,
        pltpu.SemaphoreType.DMA,
        pltpu.SemaphoreType.DMA,
    ],
)
def _edge_pass(src_hbm, dst_hbm, ee_hbm, xl_hbm, xr_hbm, shift_hbm, att_hbm,
               den_out, acc_out,
               shf_v, den_v, att_v, src0, src1, dst0, dst1, ee0, ee1,
               xl0, xl1, xr0, xr1, acc_sh, sem0, sem1, semi0, semi1):
    sid, cid = _worker_ids()
    lane = lax.iota(jnp.int32, 16)
    srcb, dstb = (src0, src1), (dst0, dst1)
    eeb, xlb, xrb, semb = (ee0, ee1), (xl0, xl1), (xr0, xr1), (sem0, sem1)
    semib = (semi0, semi1)

    def zrows(i, c):
        for j in range(_D // 16):
            ee0[i, pl.ds(16 * j, 16)] = _zero16()
        return c
    lax.fori_loop(0, _CH, zrows, 0)
    for p in range(_RPT // _CH):
        pltpu.sync_copy(ee0, acc_sh.at[pl.ds(sid * _RPT + p * _CH, _CH)])

    def zden(i, c):
        den_v[pl.ds(16 * i, 16)] = _zero16()
        return c
    lax.fori_loop(0, _NP // 16, zden, 0)

    pltpu.sync_copy(shift_hbm, shf_v)
    pltpu.sync_copy(att_hbm, att_v)
    plsc.subcore_barrier()

    chunks_per_sc = (_E // 2) // _CH
    nw, start = _chunk_split(chunks_per_sc, sid)
    ebase = cid * (_E // 2)

    def issue_idx(k, b):
        base = ebase + (start + k) * _CH
        pltpu.async_copy(src_hbm.at[pl.ds(base, _CH)], srcb[b], semib[b])
        pltpu.async_copy(dst_hbm.at[pl.ds(base, _CH)], dstb[b], semib[b])

    def wait_idx(k, b):
        base = ebase + (start + k) * _CH
        pltpu.make_async_copy(
            src_hbm.at[pl.ds(base, _CH)], srcb[b], semib[b]).wait()
        pltpu.make_async_copy(
            dst_hbm.at[pl.ds(base, _CH)], dstb[b], semib[b]).wait()

    def issue_gathers(k, b):
        base = ebase + (start + k) * _CH
        pltpu.async_copy(xl_hbm.at[srcb[b]], xlb[b], semb[b])
        pltpu.async_copy(xr_hbm.at[dstb[b]], xrb[b], semb[b])
        pltpu.async_copy(ee_hbm.at[pl.ds(base, _CH)], eeb[b], semb[b])

    issue_idx(0, 0)
    wait_idx(0, 0)
    issue_gathers(0, 0)

    @pl.when(nw > 1)
    def _():
        issue_idx(1, 1)

    def outer(kk, c):
        for b in range(2):
            k = 2 * kk + b

            @pl.when(k < nw)
            def _():
                @pl.when(k + 1 < nw)
                def _():
                    wait_idx(k + 1, 1 - b)
                    issue_gathers(k + 1, 1 - b)
                base = ebase + (start + k) * _CH
                pltpu.make_async_copy(
                    xl_hbm.at[srcb[b]], xlb[b], semb[b]).wait()
                pltpu.make_async_copy(
                    xr_hbm.at[dstb[b]], xrb[b], semb[b]).wait()
                pltpu.make_async_copy(
                    ee_hbm.at[pl.ds(base, _CH)], eeb[b], semb[b]).wait()

                def group_body(g, c2):
                    gsl = pl.ds(16 * g, 16)
                    d16 = dstb[b][gsl]
                    sh16 = plsc.load_gather(shf_v, [d16])
                    acc16 = _zero16()
                    for e in range(16):
                        ei = 16 * g + e
                        a = _zero16()
                        for j in range(_D // 16):
                            sl = pl.ds(16 * j, 16)
                            z = xlb[b][ei, sl] + xrb[b][ei, sl] + eeb[b][ei, sl]
                            z = jnp.where(z >= 0.0, z, 0.2 * z)
                            a = a + z * att_v[sl]
                        acc16 = jnp.where(lane == e, jnp.sum(a), acc16)
                    ex16 = jnp.exp(acc16 - sh16)
                    for e in range(16):
                        plsc.addupdate_scatter(den_v, [d16], ex16,
                                               mask=lane == e)
                    for e in range(16):
                        ei = 16 * g + e
                        w = jnp.sum(jnp.where(lane == e, ex16, 0.0))
                        for j in range(_D // 16):
                            sl = pl.ds(16 * j, 16)
                            xlb[b][ei, sl] = xlb[b][ei, sl] * w
                    return c2
                lax.fori_loop(0, _CH // 16, group_body, 0)

                pltpu.sync_copy(xlb[b], acc_sh.at[dstb[b]], add=True)

                @pl.when(k + 2 < nw)
                def _():
                    issue_idx(k + 2, b)
        return c
    lax.fori_loop(0, (nw + 1) // 2, outer, 0)

    pltpu.sync_copy(den_v, den_out.at[cid, sid])
    plsc.subcore_barrier()
    pltpu.sync_copy(acc_sh.at[pl.ds(sid * _RPT, _RPT)],
                    acc_out.at[cid, pl.ds(sid * _RPT, _RPT)])


_RB = 2000


def _mm2_body(x_ref, wl_ref, wr_ref, xl_ref, xr_ref):
    xb = x_ref[...]
    xl_ref[...] = jnp.dot(xb, wl_ref[...], preferred_element_type=jnp.float32)
    xr_ref[...] = jnp.dot(xb, wr_ref[...], preferred_element_type=jnp.float32)


def _mm2(x, wl, wr):
    return pl.pallas_call(
        _mm2_body,
        grid=(_N // _RB,),
        in_specs=[
            pl.BlockSpec((_RB, _D), lambda i: (i, 0)),
            pl.BlockSpec((_D, _D), lambda i: (0, 0)),
            pl.BlockSpec((_D, _D), lambda i: (0, 0)),
        ],
        out_specs=[pl.BlockSpec((_RB, _D), lambda i: (i, 0))] * 2,
        out_shape=[jax.ShapeDtypeStruct((_N, _D), jnp.float32)] * 2,
    )(x, wl, wr)


_EB = 8000


def _ee_body(ea_ref, we_ref, ee_ref):
    ee_ref[...] = jnp.dot(ea_ref[...], we_ref[...],
                          preferred_element_type=jnp.float32)


def _ee_mm(ea, we):
    return pl.pallas_call(
        _ee_body,
        grid=(_E // _EB,),
        in_specs=[
            pl.BlockSpec((_EB, _DE), lambda i: (i, 0)),
            pl.BlockSpec((_DE, _D), lambda i: (0, 0)),
        ],
        out_specs=pl.BlockSpec((_EB, _D), lambda i: (i, 0)),
        out_shape=jax.ShapeDtypeStruct((_E, _D), jnp.float32),
    )(ea, we)


def _shift_body(xl_ref, xr_ref, attr_ref, we_ref, att_ref, s_ref):
    deg = attr_ref[0, :, 16:17] + attr_ref[1, :, 16:17]
    ma = (attr_ref[0, :, :_DE] + attr_ref[1, :, :_DE]) / jnp.maximum(deg, 1.0)
    z = xl_ref[...] + xr_ref[...] + jnp.dot(
        ma, we_ref[...], preferred_element_type=jnp.float32)
    z = jnp.where(z >= 0.0, z, 0.2 * z)
    s_ref[...] = jnp.dot(z, att_ref[...], preferred_element_type=jnp.float32)


def _shift(xl, xr, attrdeg, we, att):
    s = pl.pallas_call(
        _shift_body,
        grid=(_N // _RB,),
        in_specs=[
            pl.BlockSpec((_RB, _D), lambda i: (i, 0)),
            pl.BlockSpec((_RB, _D), lambda i: (i, 0)),
            pl.BlockSpec((2, _RB, _D), lambda i: (0, i, 0)),
            pl.BlockSpec((_DE, _D), lambda i: (0, 0)),
            pl.BlockSpec((_D, 1), lambda i: (0, 0)),
        ],
        out_specs=pl.BlockSpec((_RB, 1), lambda i: (i, 0)),
        out_shape=jax.ShapeDtypeStruct((_N, 1), jnp.float32),
    )(xl, xr, attrdeg, we, att.reshape(_D, 1))
    return jnp.pad(s.reshape(_N), (0, _NP - _N))


def _combine_body(acc_ref, den_ref, xl_ref, b_ref, h_ref):
    a = acc_ref[0] + acc_ref[1] + xl_ref[...]
    dn = jnp.sum(den_ref[...], axis=1, keepdims=True) + 1.0
    h = a / dn + b_ref[...]
    h_ref[...] = jnp.maximum(h, 0.0)


def _combine(acc, den, xl, b):
    return pl.pallas_call(
        _combine_body,
        grid=(_N // _RB,),
        in_specs=[
            pl.BlockSpec((2, _RB, _D), lambda i: (0, i, 0)),
            pl.BlockSpec((_RB, 32), lambda i: (i, 0)),
            pl.BlockSpec((_RB, _D), lambda i: (i, 0)),
            pl.BlockSpec((1, _D), lambda i: (0, 0)),
        ],
        out_specs=pl.BlockSpec((_RB, _D), lambda i: (i, 0)),
        out_shape=jax.ShapeDtypeStruct((_N, _D), jnp.float32),
    )(acc, den, xl, b.reshape(1, _D))


def _pool_body(h_ref, batch_ref, wlin_ref, blin_ref, o_ref):
    b = batch_ref[...]
    g = lax.broadcasted_iota(jnp.int32, (_NG, 1), 0)
    oh = (b == g).astype(jnp.float32)
    cnt = jnp.sum(oh, axis=1, keepdims=True)
    ps = jnp.dot(oh, h_ref[...], preferred_element_type=jnp.float32)
    pooled = ps / jnp.maximum(cnt, 1.0)
    o_ref[...] = jnp.dot(pooled, wlin_ref[...],
                         preferred_element_type=jnp.float32) + blin_ref[...]


def _pool(h, batch, wlin, blin):
    nc = wlin.shape[1]
    return pl.pallas_call(
        _pool_body,
        out_shape=jax.ShapeDtypeStruct((_NG, nc), jnp.float32),
    )(h, batch.reshape(1, _N), wlin, blin.reshape(1, nc))


def _layer(x, src, dst, ea, attrdeg, wl, wr, we, att, b):
    xl, xr = _mm2(x, wl, wr)
    ee = _ee_mm(ea, we)
    sh = _shift(xl, xr, attrdeg, we, att)
    den, acc = _edge_pass(src, dst, ee, xl, xr, sh, att)
    den32 = den.reshape(32, _NP)[:, :_N].T
    return _combine(acc, den32, xl, b)


def kernel(x, edge_index, batch, dropout, edge_attr, Wl1, Wr1, We1, att1, b1,
           Wl2, Wr2, We2, att2, b2, Wlin, blin):
    src = edge_index[0]
    dst = edge_index[1]
    attrdeg = _deg_attr_pass(dst, edge_attr)
    h = _layer(x, src, dst, edge_attr, attrdeg, Wl1, Wr1, We1, att1, b1)
    h = _layer(h, src, dst, edge_attr, attrdeg, Wl2, Wr2, We2, att2, b2)
    return _pool(h, batch, Wlin, blin)
